# baseline (speedup 1.0000x reference)
import jax
import jax.numpy as jnp
from jax.experimental import pallas as pl


TILE_N = 512


def _chamfer_body(a_ref, b_ref, dist1_ref, dist2_ref):
    i = pl.program_id(1)
    d = jax.lax.dot_general(
        a_ref[0], b_ref[0], (((1,), (1,)), ((), ())),
        preferred_element_type=jnp.float32,
    )
    dist1_ref[0, :, :] = jnp.min(d, axis=1, keepdims=True)
    partial = jnp.min(d, axis=0, keepdims=True)

    @pl.when(i == 0)
    def _init():
        dist2_ref[0, :, :] = partial

    @pl.when(i > 0)
    def _acc():
        dist2_ref[0, :, :] = jnp.minimum(dist2_ref[0, :, :], partial)


def _split3_bf16(s):
    hi = s.astype(jnp.bfloat16).astype(jnp.float32)
    r1 = s - hi
    mid = r1.astype(jnp.bfloat16).astype(jnp.float32)
    lo = r1 - mid
    return hi, mid, lo


@jax.jit
def kernel(input1, input2):
    b, n, _ = input1.shape
    m = input2.shape[1]
    sq1 = jnp.sum(input1 * input1, axis=-1)
    sq2 = jnp.sum(input2 * input2, axis=-1)
    s1h, s1m, s1l = _split3_bf16(sq1)
    s2h, s2m, s2l = _split3_bf16(sq2)
    ones1 = jnp.ones((b, n, 3), jnp.float32)
    ones2 = jnp.ones((b, m, 3), jnp.float32)
    zeros1 = jnp.zeros((b, n, 7), jnp.float32)
    zeros2 = jnp.zeros((b, m, 7), jnp.float32)
    a = jnp.concatenate(
        [-2.0 * input1, s1h[..., None], s1m[..., None], s1l[..., None],
         ones1, zeros1], axis=-1).astype(jnp.bfloat16)
    bb = jnp.concatenate(
        [input2, ones2, s2h[..., None], s2m[..., None], s2l[..., None],
         zeros2], axis=-1).astype(jnp.bfloat16)

    grid = (b, n // TILE_N)
    dist1, dist2 = pl.pallas_call(
        _chamfer_body,
        grid=grid,
        in_specs=[
            pl.BlockSpec((1, TILE_N, 16), lambda bi, i: (bi, i, 0)),
            pl.BlockSpec((1, m, 16), lambda bi, i: (bi, 0, 0)),
        ],
        out_specs=[
            pl.BlockSpec((1, TILE_N, 1), lambda bi, i: (bi, i, 0)),
            pl.BlockSpec((1, 1, m), lambda bi, i: (bi, 0, 0)),
        ],
        out_shape=[
            jax.ShapeDtypeStruct((b, n, 1), jnp.float32),
            jax.ShapeDtypeStruct((b, 1, m), jnp.float32),
        ],
    )(a, bb)
    return dist1[:, :, 0], dist2[:, 0, :]


# speedup vs baseline: 1.9430x; 1.9430x over previous
import jax
import jax.numpy as jnp
from jax.experimental import pallas as pl
from jax.experimental.pallas import tpu as pltpu


TILE_N = 512


def _sq_split3(x):
    y = x * x
    s = (y[:, 0:1] + y[:, 1:2]) + y[:, 2:3]
    hi = s.astype(jnp.bfloat16).astype(jnp.float32)
    r1 = s - hi
    mid = r1.astype(jnp.bfloat16).astype(jnp.float32)
    lo = r1 - mid
    return hi, mid, lo


def _chamfer_body(x1_ref, x2_ref, dist1_ref, dist2_ref, bmat_ref):
    i = pl.program_id(1)

    @pl.when(i == 0)
    def _build_b():
        x2 = x2_ref[0]
        hi, mid, lo = _sq_split3(x2)
        ones = jnp.ones_like(hi)
        bmat = jnp.concatenate([x2, ones, ones, ones, hi, mid, lo], axis=1)
        bmat_ref[...] = bmat.astype(jnp.bfloat16)

    x1 = x1_ref[0]
    hi, mid, lo = _sq_split3(x1)
    ones = jnp.ones_like(hi)
    amat = jnp.concatenate([-2.0 * x1, hi, mid, lo, ones, ones, ones], axis=1)
    d = jax.lax.dot_general(
        amat.astype(jnp.bfloat16), bmat_ref[...],
        (((1,), (1,)), ((), ())),
        preferred_element_type=jnp.float32,
    )
    dist1_ref[0, :, :] = jnp.min(d, axis=1, keepdims=True)
    partial = jnp.min(d, axis=0, keepdims=True)

    @pl.when(i == 0)
    def _init():
        dist2_ref[0, :, :] = partial

    @pl.when(i > 0)
    def _acc():
        dist2_ref[0, :, :] = jnp.minimum(dist2_ref[0, :, :], partial)


@jax.jit
def kernel(input1, input2):
    b, n, _ = input1.shape
    m = input2.shape[1]
    grid = (b, n // TILE_N)
    dist1, dist2 = pl.pallas_call(
        _chamfer_body,
        grid=grid,
        in_specs=[
            pl.BlockSpec((1, TILE_N, 3), lambda bi, i: (bi, i, 0)),
            pl.BlockSpec((1, m, 3), lambda bi, i: (bi, 0, 0)),
        ],
        out_specs=[
            pl.BlockSpec((1, TILE_N, 1), lambda bi, i: (bi, i, 0)),
            pl.BlockSpec((1, 1, m), lambda bi, i: (bi, 0, 0)),
        ],
        out_shape=[
            jax.ShapeDtypeStruct((b, n, 1), jnp.float32),
            jax.ShapeDtypeStruct((b, 1, m), jnp.float32),
        ],
        scratch_shapes=[pltpu.VMEM((m, 9), jnp.bfloat16)],
    )(input1, input2)
    return dist1[:, :, 0], dist2[:, 0, :]


# speedup vs baseline: 1.9584x; 1.0079x over previous
import jax
import jax.numpy as jnp
from jax.experimental import pallas as pl
from jax.experimental.pallas import tpu as pltpu


TILE_N = 512


def _sq_split3(x):
    y = x * x
    s = (y[:, 0:1] + y[:, 1:2]) + y[:, 2:3]
    hi = s.astype(jnp.bfloat16).astype(jnp.float32)
    r1 = s - hi
    mid = r1.astype(jnp.bfloat16).astype(jnp.float32)
    lo = r1 - mid
    return hi, mid, lo


def _chamfer_body(x1_ref, x2_ref, dist1_ref, dist2_ref, bmat_ref):
    i = pl.program_id(1)

    @pl.when(i == 0)
    def _build_b():
        x2 = x2_ref[0]
        hi, mid, lo = _sq_split3(x2)
        ones = jnp.ones_like(hi)
        bmat_ref[...] = jnp.concatenate(
            [x2, ones, ones, ones, hi, mid, lo], axis=1)

    x1 = x1_ref[0]
    hi, mid, lo = _sq_split3(x1)
    ones = jnp.ones_like(hi)
    amat = jnp.concatenate([-2.0 * x1, hi, mid, lo, ones, ones, ones], axis=1)
    d = jax.lax.dot_general(
        amat, bmat_ref[...],
        (((1,), (1,)), ((), ())),
        preferred_element_type=jnp.float32,
    )
    dist1_ref[0, :, :] = jnp.min(d, axis=1, keepdims=True)
    partial = jnp.min(d, axis=0, keepdims=True)

    @pl.when(i == 0)
    def _init():
        dist2_ref[0, :, :] = partial

    @pl.when(i > 0)
    def _acc():
        dist2_ref[0, :, :] = jnp.minimum(dist2_ref[0, :, :], partial)


@jax.jit
def kernel(input1, input2):
    b, n, _ = input1.shape
    m = input2.shape[1]
    grid = (b, n // TILE_N)
    dist1, dist2 = pl.pallas_call(
        _chamfer_body,
        grid=grid,
        in_specs=[
            pl.BlockSpec((1, TILE_N, 3), lambda bi, i: (bi, i, 0)),
            pl.BlockSpec((1, m, 3), lambda bi, i: (bi, 0, 0)),
        ],
        out_specs=[
            pl.BlockSpec((1, TILE_N, 1), lambda bi, i: (bi, i, 0)),
            pl.BlockSpec((1, 1, m), lambda bi, i: (bi, 0, 0)),
        ],
        out_shape=[
            jax.ShapeDtypeStruct((b, n, 1), jnp.float32),
            jax.ShapeDtypeStruct((b, 1, m), jnp.float32),
        ],
        scratch_shapes=[pltpu.VMEM((m, 9), jnp.float32)],
    )(input1, input2)
    return dist1[:, :, 0], dist2[:, 0, :]


# speedup vs baseline: 2.2238x; 1.1355x over previous
import jax
import jax.numpy as jnp
from jax.experimental import pallas as pl
from jax.experimental.pallas import tpu as pltpu


TILE_N = 512


def _chamfer_body(a_ref, b_ref, dist1_ref, dist2_ref, amat_ref, bmat_ref):
    i = pl.program_id(1)

    @pl.when(i == 0)
    def _build_b():
        bmat_ref[...] = jnp.transpose(b_ref[0], (1, 0))

    amat_ref[...] = jnp.transpose(a_ref[0], (1, 0))
    d = jax.lax.dot_general(
        amat_ref[...], bmat_ref[...],
        (((1,), (1,)), ((), ())),
        preferred_element_type=jnp.float32,
    )
    dist1_ref[0, :, :] = jnp.min(d, axis=1, keepdims=True)
    partial = jnp.min(d, axis=0, keepdims=True)

    @pl.when(i == 0)
    def _init():
        dist2_ref[0, :, :] = partial

    @pl.when(i > 0)
    def _acc():
        dist2_ref[0, :, :] = jnp.minimum(dist2_ref[0, :, :], partial)


def _augment_t(xyz, scale_xyz, sq_first):
    t = jnp.transpose(xyz, (0, 2, 1))
    y = t * t
    s = (y[:, 0:1, :] + y[:, 1:2, :]) + y[:, 2:3, :]
    hi = jax.lax.reduce_precision(s, exponent_bits=8, mantissa_bits=7)
    r1 = s - hi
    mid = jax.lax.reduce_precision(r1, exponent_bits=8, mantissa_bits=7)
    lo = r1 - mid
    ones = jnp.ones_like(s)
    parts = [scale_xyz * t]
    if sq_first:
        parts += [hi, mid, lo, ones, ones, ones]
    else:
        parts += [ones, ones, ones, hi, mid, lo]
    return jnp.concatenate(parts, axis=1)


@jax.jit
def kernel(input1, input2):
    b, n, _ = input1.shape
    m = input2.shape[1]
    at = _augment_t(input1, -2.0, True)
    bt = _augment_t(input2, 1.0, False)
    grid = (b, n // TILE_N)
    dist1, dist2 = pl.pallas_call(
        _chamfer_body,
        grid=grid,
        in_specs=[
            pl.BlockSpec((1, 9, TILE_N), lambda bi, i: (bi, 0, i)),
            pl.BlockSpec((1, 9, m), lambda bi, i: (bi, 0, 0)),
        ],
        out_specs=[
            pl.BlockSpec((1, TILE_N, 1), lambda bi, i: (bi, i, 0)),
            pl.BlockSpec((1, 1, m), lambda bi, i: (bi, 0, 0)),
        ],
        out_shape=[
            jax.ShapeDtypeStruct((b, n, 1), jnp.float32),
            jax.ShapeDtypeStruct((b, 1, m), jnp.float32),
        ],
        scratch_shapes=[
            pltpu.VMEM((TILE_N, 9), jnp.float32),
            pltpu.VMEM((m, 9), jnp.float32),
        ],
    )(at, bt)
    return dist1[:, :, 0], dist2[:, 0, :]


# speedup vs baseline: 2.7232x; 1.2246x over previous
import jax
import jax.numpy as jnp
from jax.experimental import pallas as pl
from jax.experimental.pallas import tpu as pltpu


TILE_N = 512


def _augment_t(t, scale_xyz, sq_first):
    y = t * t
    s = (y[0:1, :] + y[1:2, :]) + y[2:3, :]
    hi = s.astype(jnp.bfloat16).astype(jnp.float32)
    r1 = s - hi
    mid = r1.astype(jnp.bfloat16).astype(jnp.float32)
    lo = r1 - mid
    ones = jnp.ones_like(s)
    parts = [scale_xyz * t]
    if sq_first:
        parts += [hi, mid, lo, ones, ones, ones]
    else:
        parts += [ones, ones, ones, hi, mid, lo]
    return jnp.concatenate(parts, axis=0)


def _chamfer_body(x1_ref, x2_ref, dist1_ref, dist2_ref, amat_ref, bmat_ref):
    i = pl.program_id(1)

    @pl.when(i == 0)
    def _build_b():
        b9 = _augment_t(x2_ref[0], 1.0, False)
        bmat_ref[...] = jnp.transpose(b9, (1, 0)).astype(jnp.bfloat16)

    a9 = _augment_t(x1_ref[0], -2.0, True)
    amat_ref[...] = jnp.transpose(a9, (1, 0)).astype(jnp.bfloat16)
    d = jax.lax.dot_general(
        amat_ref[...], bmat_ref[...],
        (((1,), (1,)), ((), ())),
        preferred_element_type=jnp.float32,
    )
    rowmin = jnp.min(d, axis=1, keepdims=True)
    dist1_ref[0, :, :] = jnp.transpose(rowmin, (1, 0))
    partial = jnp.min(d, axis=0, keepdims=True)

    @pl.when(i == 0)
    def _init():
        dist2_ref[0, :, :] = partial

    @pl.when(i > 0)
    def _acc():
        dist2_ref[0, :, :] = jnp.minimum(dist2_ref[0, :, :], partial)


@jax.jit
def kernel(input1, input2):
    b, n, _ = input1.shape
    m = input2.shape[1]
    x1t = jnp.transpose(input1, (0, 2, 1))
    x2t = jnp.transpose(input2, (0, 2, 1))
    grid = (b, n // TILE_N)
    dist1, dist2 = pl.pallas_call(
        _chamfer_body,
        grid=grid,
        in_specs=[
            pl.BlockSpec((1, 3, TILE_N), lambda bi, i: (bi, 0, i)),
            pl.BlockSpec((1, 3, m), lambda bi, i: (bi, 0, 0)),
        ],
        out_specs=[
            pl.BlockSpec((1, 1, TILE_N), lambda bi, i: (bi, 0, i)),
            pl.BlockSpec((1, 1, m), lambda bi, i: (bi, 0, 0)),
        ],
        out_shape=[
            jax.ShapeDtypeStruct((b, 1, n), jnp.float32),
            jax.ShapeDtypeStruct((b, 1, m), jnp.float32),
        ],
        scratch_shapes=[
            pltpu.VMEM((TILE_N, 9), jnp.bfloat16),
            pltpu.VMEM((m, 9), jnp.bfloat16),
        ],
    )(x1t, x2t)
    return dist1[:, 0, :], dist2[:, 0, :]


# speedup vs baseline: 3.0542x; 1.1216x over previous
import jax
import jax.numpy as jnp
from jax.experimental import pallas as pl
from jax.experimental.pallas import tpu as pltpu


TILE_N = 1024
M_CHUNKS = 2


def _augment_t(t, scale_xyz, sq_first):
    y = t * t
    s = (y[0:1, :] + y[1:2, :]) + y[2:3, :]
    hi = s.astype(jnp.bfloat16).astype(jnp.float32)
    r1 = s - hi
    mid = r1.astype(jnp.bfloat16).astype(jnp.float32)
    lo = r1 - mid
    ones = jnp.ones_like(s)
    parts = [scale_xyz * t]
    if sq_first:
        parts += [hi, mid, lo, ones, ones, ones]
    else:
        parts += [ones, ones, ones, hi, mid, lo]
    return jnp.concatenate(parts, axis=0)


def _chamfer_body(x1_ref, x2_ref, dist1_ref, dist2_ref, amat_ref, bmat_ref):
    i = pl.program_id(1)

    @pl.when(i == 0)
    def _build_b():
        b9 = _augment_t(x2_ref[0], 1.0, False)
        bmat_ref[...] = jnp.transpose(b9, (1, 0)).astype(jnp.bfloat16)

    a9 = _augment_t(x1_ref[0], -2.0, True)
    amat_ref[...] = jnp.transpose(a9, (1, 0)).astype(jnp.bfloat16)
    amat = amat_ref[...]
    m = bmat_ref.shape[0]
    mc = m // M_CHUNKS
    rowmin = None
    colmins = []
    for c in range(M_CHUNKS):
        d = jax.lax.dot_general(
            amat, bmat_ref[c * mc:(c + 1) * mc, :],
            (((1,), (1,)), ((), ())),
            preferred_element_type=jnp.float32,
        )
        rm = jnp.min(d, axis=1, keepdims=True)
        rowmin = rm if rowmin is None else jnp.minimum(rowmin, rm)
        colmins.append(jnp.min(d, axis=0, keepdims=True))
    dist1_ref[0, :, :] = jnp.transpose(rowmin, (1, 0))
    partial = jnp.concatenate(colmins, axis=1)

    @pl.when(i == 0)
    def _init():
        dist2_ref[0, :, :] = partial

    @pl.when(i > 0)
    def _acc():
        dist2_ref[0, :, :] = jnp.minimum(dist2_ref[0, :, :], partial)


@jax.jit
def kernel(input1, input2):
    b, n, _ = input1.shape
    m = input2.shape[1]
    x1t = jnp.transpose(input1, (0, 2, 1))
    x2t = jnp.transpose(input2, (0, 2, 1))
    grid = (b, n // TILE_N)
    dist1, dist2 = pl.pallas_call(
        _chamfer_body,
        grid=grid,
        in_specs=[
            pl.BlockSpec((1, 3, TILE_N), lambda bi, i: (bi, 0, i)),
            pl.BlockSpec((1, 3, m), lambda bi, i: (bi, 0, 0)),
        ],
        out_specs=[
            pl.BlockSpec((1, 1, TILE_N), lambda bi, i: (bi, 0, i)),
            pl.BlockSpec((1, 1, m), lambda bi, i: (bi, 0, 0)),
        ],
        out_shape=[
            jax.ShapeDtypeStruct((b, 1, n), jnp.float32),
            jax.ShapeDtypeStruct((b, 1, m), jnp.float32),
        ],
        scratch_shapes=[
            pltpu.VMEM((TILE_N, 9), jnp.bfloat16),
            pltpu.VMEM((m, 9), jnp.bfloat16),
        ],
    )(x1t, x2t)
    return dist1[:, 0, :], dist2[:, 0, :]
